# baseline (speedup 1.0000x reference)
import jax
import jax.numpy as jnp
from jax.experimental import pallas as pl

SUPPORT_RANGE = 300
EPS = 0.001
NUM_BINS = 2 * SUPPORT_RANGE + 1

BLOCK_ROWS = 512


def _mu_zero_block(logits_ref, w_ref, out_ref):
    logits = logits_ref[...]
    rows = logits.shape[0]

    m = jnp.max(logits, axis=-1, keepdims=True)
    e = jnp.exp(logits - m)
    sums = jax.lax.dot_general(
        e,
        w_ref[...],
        dimension_numbers=(((1,), (0,)), ((), ())),
        preferred_element_type=jnp.float32,
        precision=jax.lax.Precision.HIGHEST,
    )
    x = sums[:, 1] / sums[:, 0]

    scalar = jnp.sign(x) * (
        ((jnp.sqrt(1.0 + 4.0 * EPS * (jnp.abs(x) + 1.0 + EPS)) - 1.0) / (2.0 * EPS))
        ** 2
        - 1.0
    )

    y = jnp.sign(scalar) * (jnp.sqrt(jnp.abs(scalar) + 1.0) - 1.0) + EPS * scalar
    y = jnp.clip(y, -float(SUPPORT_RANGE), float(SUPPORT_RANGE))

    bins = jax.lax.broadcasted_iota(jnp.int32, (rows, NUM_BINS), 1)
    support = bins.astype(jnp.float32) - float(SUPPORT_RANGE)
    out_ref[...] = jnp.maximum(0.0, 1.0 - jnp.abs(support - y[:, None]))


@jax.jit
def kernel(logits):
    n_rows = logits.shape[0]
    support = jnp.arange(-SUPPORT_RANGE, SUPPORT_RANGE + 1, dtype=jnp.float32)
    w = jnp.stack([jnp.ones_like(support), support], axis=1)
    grid = (n_rows // BLOCK_ROWS,)
    return pl.pallas_call(
        _mu_zero_block,
        grid=grid,
        in_specs=[
            pl.BlockSpec((BLOCK_ROWS, NUM_BINS), lambda i: (i, 0)),
            pl.BlockSpec((NUM_BINS, 2), lambda i: (0, 0)),
        ],
        out_specs=pl.BlockSpec((BLOCK_ROWS, NUM_BINS), lambda i: (i, 0)),
        out_shape=jax.ShapeDtypeStruct((n_rows, NUM_BINS), jnp.float32),
    )(logits, w)


# speedup vs baseline: 1.1332x; 1.1332x over previous
import jax
import jax.numpy as jnp
from jax.experimental import pallas as pl

SUPPORT_RANGE = 300
EPS = 0.001
NUM_BINS = 2 * SUPPORT_RANGE + 1

BLOCK_ROWS = 512


def _mu_zero_block(logits_ref, out_ref):
    logits = logits_ref[...]
    rows = logits.shape[0]

    bins = jax.lax.broadcasted_iota(jnp.int32, (rows, NUM_BINS), 1)
    support = bins.astype(jnp.float32) - float(SUPPORT_RANGE)

    m = jnp.max(logits, axis=-1, keepdims=True)
    e = jnp.exp(logits - m)
    x = jnp.sum(e * support, axis=-1) / jnp.sum(e, axis=-1)

    scalar = jnp.sign(x) * (
        ((jnp.sqrt(1.0 + 4.0 * EPS * (jnp.abs(x) + 1.0 + EPS)) - 1.0) / (2.0 * EPS))
        ** 2
        - 1.0
    )

    y = jnp.sign(scalar) * (jnp.sqrt(jnp.abs(scalar) + 1.0) - 1.0) + EPS * scalar
    y = jnp.clip(y, -float(SUPPORT_RANGE), float(SUPPORT_RANGE))

    out_ref[...] = jnp.maximum(0.0, 1.0 - jnp.abs(support - y[:, None]))


@jax.jit
def kernel(logits):
    n_rows = logits.shape[0]
    grid = (n_rows // BLOCK_ROWS,)
    return pl.pallas_call(
        _mu_zero_block,
        grid=grid,
        in_specs=[pl.BlockSpec((BLOCK_ROWS, NUM_BINS), lambda i: (i, 0))],
        out_specs=pl.BlockSpec((BLOCK_ROWS, NUM_BINS), lambda i: (i, 0)),
        out_shape=jax.ShapeDtypeStruct((n_rows, NUM_BINS), jnp.float32),
    )(logits)


# speedup vs baseline: 1.2604x; 1.1122x over previous
import jax
import jax.numpy as jnp
from jax.experimental import pallas as pl

SUPPORT_RANGE = 300
EPS = 0.001
NUM_BINS = 2 * SUPPORT_RANGE + 1

BLOCK_ROWS = 2048


def _mu_zero_block(logits_ref, out_ref):
    logits = logits_ref[...]
    rows = logits.shape[0]

    bins = jax.lax.broadcasted_iota(jnp.int32, (rows, NUM_BINS), 1)
    support = bins.astype(jnp.float32) - float(SUPPORT_RANGE)

    m = jnp.max(logits, axis=-1, keepdims=True)
    e = jnp.exp(logits - m)
    x = jnp.sum(e * support, axis=-1) / jnp.sum(e, axis=-1)

    scalar = jnp.sign(x) * (
        ((jnp.sqrt(1.0 + 4.0 * EPS * (jnp.abs(x) + 1.0 + EPS)) - 1.0) / (2.0 * EPS))
        ** 2
        - 1.0
    )

    y = jnp.sign(scalar) * (jnp.sqrt(jnp.abs(scalar) + 1.0) - 1.0) + EPS * scalar
    y = jnp.clip(y, -float(SUPPORT_RANGE), float(SUPPORT_RANGE))

    out_ref[...] = jnp.maximum(0.0, 1.0 - jnp.abs(support - y[:, None]))


@jax.jit
def kernel(logits):
    n_rows = logits.shape[0]
    grid = (n_rows // BLOCK_ROWS,)
    return pl.pallas_call(
        _mu_zero_block,
        grid=grid,
        in_specs=[pl.BlockSpec((BLOCK_ROWS, NUM_BINS), lambda i: (i, 0))],
        out_specs=pl.BlockSpec((BLOCK_ROWS, NUM_BINS), lambda i: (i, 0)),
        out_shape=jax.ShapeDtypeStruct((n_rows, NUM_BINS), jnp.float32),
    )(logits)
